# baseline (speedup 1.0000x reference)
import functools

import jax
import jax.numpy as jnp
from jax import lax
from jax.experimental import pallas as pl
from jax.experimental.pallas import tpu as pltpu
from jax.experimental.pallas import tpu_sc as plsc

N = 10000
D = 256
E = 160000
R = 3
NC = 2
NS = 16
NCH = 80
EPT = NCH * 128
EPAD = EPT * NS
NH = N + 112
RPT = NH // NS
BN = 400
NBLK = N // BN
NPAIR = NCH // 2

_mesh = plsc.VectorSubcoreMesh(
    core_axis_name="c", subcore_axis_name="s", num_cores=NC, num_subcores=NS)

HB = R * NH


@functools.partial(
    pl.kernel,
    out_type=jax.ShapeDtypeStruct((NC * NS, HB), jnp.float32),
    mesh=_mesh,
    scratch_types=[
        pltpu.VMEM((HB,), jnp.float32),
        pltpu.VMEM((NCH // 2, 128), jnp.int32),
    ],
    compiler_params=pltpu.CompilerParams(needs_layout_passes=False),
)
def _deg_kernel(dst_hbm, out_hbm, hist, dstb):
    c = lax.axis_index("c")
    t = lax.axis_index("s")
    wid = c * NS + t
    zeros16 = jnp.zeros((16,), jnp.float32)
    ones16 = jnp.ones((16,), jnp.float32)

    def zbody(i, _):
        hist[pl.ds(i * 16, 16)] = zeros16
        return 0

    lax.fori_loop(0, HB // 16, zbody, 0)
    for j in range(R):
        pltpu.sync_copy(dst_hbm.at[j, t, pl.ds(c * (NCH // 2), NCH // 2)],
                        dstb)

        def body(r, _):
            for k in range(8):
                idx = dstb[r, pl.ds(k * 16, 16)]
                plsc.addupdate_scatter(hist, [idx + j * NH], ones16)
            return 0

        lax.fori_loop(0, NCH // 2, body, 0)
    pltpu.sync_copy(hist, out_hbm.at[wid])


@functools.partial(
    pl.kernel,
    out_type=jax.ShapeDtypeStruct((R, NC, NH, 128), jnp.float32),
    mesh=_mesh,
    scratch_types=[
        pltpu.VMEM_SHARED((NH, 128), jnp.float32),
        pltpu.VMEM((NCH // 2, 128), jnp.int32),
        pltpu.VMEM((NCH // 2, 128), jnp.int32),
        pltpu.VMEM((64, 128), jnp.float32),
        pltpu.VMEM((64, 128), jnp.float32),
        pltpu.VMEM((64, 128), jnp.float32),
        pltpu.VMEM((64, 128), jnp.float32),
        pltpu.SemaphoreType.DMA,
        pltpu.SemaphoreType.DMA,
        pltpu.SemaphoreType.DMA,
        pltpu.SemaphoreType.DMA,
    ],
    compiler_params=pltpu.CompilerParams(needs_layout_passes=False),
)
def _conv_kernel(h_hbm, srcadj_hbm, dst_hbm, z_hbm, s_hbm,
                 acc, srcb, dstb, r0, r1b, r2, r3, s0, s1b, s2, s3):
    c = lax.axis_index("c")
    t = lax.axis_index("s")
    base = t * RPT
    bufs = [(r0, s0), (r1b, s1b), (r2, s2), (r3, s3)]
    for j in range(R):
        for k in range(RPT // 128):
            pltpu.sync_copy(z_hbm, acc.at[pl.ds(base + 128 * k, 128)])
        rem = RPT % 128
        if rem:
            pltpu.sync_copy(z_hbm.at[pl.ds(0, rem)],
                            acc.at[pl.ds(base + RPT - rem, rem)])
        plsc.subcore_barrier()

        def gissue(sc, buf, sem):
            pltpu.async_copy(
                h_hbm.at[srcb.at[sc // 2, pl.ds((sc % 2) * 64, 64)]],
                buf, sem)

        def gwait(buf, sem):
            pltpu.make_async_copy(
                h_hbm.at[srcb.at[0, pl.ds(0, 64)]], buf, sem).wait()

        def scat(sc, buf):
            pltpu.sync_copy(
                buf, acc.at[dstb.at[sc // 2, pl.ds((sc % 2) * 64, 64)]],
                add=True)

        NSC = NCH
        for half in range(2):
            pltpu.sync_copy(
                srcadj_hbm.at[c, j, t, pl.ds(half * (NCH // 2), NCH // 2)],
                srcb)
            pltpu.sync_copy(
                dst_hbm.at[j, t, pl.ds(half * (NCH // 2), NCH // 2)], dstb)
            for b in range(4):
                gissue(b, bufs[b][0], bufs[b][1])

            def quad(i, _):
                q0 = 4 * i
                for b in range(4):
                    gwait(bufs[b][0], bufs[b][1])
                    scat(q0 + b, bufs[b][0])

                    @pl.when(i < NSC // 4 - 1)
                    def _():
                        gissue(q0 + 4 + b, bufs[b][0], bufs[b][1])
                return 0

            lax.fori_loop(0, NSC // 4, quad, 0)
        plsc.subcore_barrier()
        pltpu.sync_copy(acc.at[pl.ds(base, RPT)],
                        s_hbm.at[j, c, pl.ds(base, RPT)])
        plsc.subcore_barrier()


def _rel_body(rel_ref, lw_ref, lb_ref, r1_ref, fr_ref):
    r1 = jnp.dot(rel_ref[...], lw_ref[0].T,
                 preferred_element_type=jnp.float32) + lb_ref[0]
    r1_ref[...] = r1.reshape(R, 1, D)
    fr_ref[...] = jnp.dot(r1, lw_ref[1].T,
                          preferred_element_type=jnp.float32) + lb_ref[1]


def _rel_call(rel_emb, lin_W, lin_b):
    return pl.pallas_call(
        _rel_body,
        out_shape=(
            jax.ShapeDtypeStruct((R, 1, D), jnp.float32),
            jax.ShapeDtypeStruct((R, D), jnp.float32),
        ),
    )(rel_emb, lin_W, lin_b)


def _mm1_body(f_ref, rel_ref, w_ref, part_ref, out_ref, dinv_ref):
    x = f_ref[...] * rel_ref[0]
    deg = jnp.sum(part_ref[:, 0, :, :], axis=0) + 1.0
    dv = lax.rsqrt(deg)
    dinv_ref[0] = dv
    w = w_ref[0]
    out_ref[0, 0] = jnp.dot(x, w[:, :128],
                            preferred_element_type=jnp.float32) * dv
    out_ref[0, 1] = jnp.dot(x, w[:, 128:],
                            preferred_element_type=jnp.float32) * dv


def _mm1_call(features, rel_emb, gcn_W, part):
    return pl.pallas_call(
        _mm1_body,
        grid=(R, NBLK),
        in_specs=[
            pl.BlockSpec((BN, D), lambda j, nb: (nb, 0)),
            pl.BlockSpec((1, 1, D), lambda j, nb: (j, 0, 0)),
            pl.BlockSpec((1, D, D), lambda j, nb: (0, 0, 0)),
            pl.BlockSpec((NC * NS, 1, BN, 1), lambda j, nb: (0, j, nb, 0)),
        ],
        out_specs=(
            pl.BlockSpec((1, NC, BN, 128), lambda j, nb: (j, 0, nb, 0)),
            pl.BlockSpec((1, BN, 1), lambda j, nb: (j, nb, 0)),
        ),
        out_shape=(
            jax.ShapeDtypeStruct((R, NC, NH, 128), jnp.float32),
            jax.ShapeDtypeStruct((R, NH, 1), jnp.float32),
        ),
    )(features, rel_emb, gcn_W, part)


def _stats_body(s_ref, h_ref, dinv_ref, out_ref):
    nb = pl.program_id(1)
    dv = dinv_ref[0]
    acc = []
    for cc in range(NC):
        tt = dv * (s_ref[0, cc] + h_ref[0, cc])
        acc.append(jnp.stack([jnp.sum(tt, axis=0), jnp.sum(tt * tt, axis=0)]))
    v = jnp.stack(acc)

    @pl.when(nb == 0)
    def _():
        out_ref[0] = v

    @pl.when(nb > 0)
    def _():
        out_ref[0] += v


def _stats_call(s, h, dinv):
    return pl.pallas_call(
        _stats_body,
        grid=(R, NBLK),
        in_specs=[
            pl.BlockSpec((1, NC, BN, 128), lambda j, nb: (j, 0, nb, 0)),
            pl.BlockSpec((1, NC, BN, 128), lambda j, nb: (j, 0, nb, 0)),
            pl.BlockSpec((1, BN, 1), lambda j, nb: (j, nb, 0)),
        ],
        out_specs=pl.BlockSpec((1, NC, 2, 128), lambda j, nb: (j, 0, 0, 0)),
        out_shape=jax.ShapeDtypeStruct((R, NC, 2, 128), jnp.float32),
    )(s, h, dinv)


def _mm2_body(f_ref, s_ref, h_ref, dinv_ref, st_ref, g_ref, be_ref,
              r1_ref, w_ref, out_ref):
    dv = dinv_ref[0]
    t = jnp.concatenate(
        [dv * (s_ref[0, cc] + h_ref[0, cc]) for cc in range(NC)], axis=1)
    ssum = jnp.concatenate([st_ref[0, cc, 0] for cc in range(NC)])
    ssq = jnp.concatenate([st_ref[0, cc, 1] for cc in range(NC)])
    mu = ssum / N
    var = ssq / N - mu * mu
    bh = (t - mu[None, :]) * lax.rsqrt(var + 1e-5)[None, :]
    bh = bh * g_ref[0][None, :] + be_ref[0][None, :]
    x1 = (f_ref[...] + jnp.where(bh >= 0.0, bh, 0.01 * bh)) * r1_ref[0]
    w = w_ref[0]
    out_ref[0, 0] = jnp.dot(x1, w[:, :128],
                            preferred_element_type=jnp.float32) * dv
    out_ref[0, 1] = jnp.dot(x1, w[:, 128:],
                            preferred_element_type=jnp.float32) * dv


def _mm2_call(features, s, h, dinv, stats, bn_gamma, bn_beta, r1, gcn_W):
    return pl.pallas_call(
        _mm2_body,
        grid=(R, NBLK),
        in_specs=[
            pl.BlockSpec((BN, D), lambda j, nb: (nb, 0)),
            pl.BlockSpec((1, NC, BN, 128), lambda j, nb: (j, 0, nb, 0)),
            pl.BlockSpec((1, NC, BN, 128), lambda j, nb: (j, 0, nb, 0)),
            pl.BlockSpec((1, BN, 1), lambda j, nb: (j, nb, 0)),
            pl.BlockSpec((1, NC, 2, 128), lambda j, nb: (j, 0, 0, 0)),
            pl.BlockSpec((1, D), lambda j, nb: (0, 0)),
            pl.BlockSpec((1, D), lambda j, nb: (0, 0)),
            pl.BlockSpec((1, 1, D), lambda j, nb: (j, 0, 0)),
            pl.BlockSpec((1, D, D), lambda j, nb: (1, 0, 0)),
        ],
        out_specs=pl.BlockSpec((1, NC, BN, 128), lambda j, nb: (j, 0, nb, 0)),
        out_shape=jax.ShapeDtypeStruct((R, NC, NH, 128), jnp.float32),
    )(features, s, h, dinv, stats, bn_gamma, bn_beta, r1, gcn_W)


def _final_body(s_ref, h_ref, dinv_ref, b_ref, out_ref):
    dv = dinv_ref[0]
    t = jnp.concatenate(
        [dv * (s_ref[0, cc] + h_ref[0, cc]) for cc in range(NC)], axis=1)
    out_ref[0] = t + b_ref[0]


def _final_call(s, h, dinv, gcn_b):
    return pl.pallas_call(
        _final_body,
        grid=(R, NBLK),
        in_specs=[
            pl.BlockSpec((1, NC, BN, 128), lambda j, nb: (j, 0, nb, 0)),
            pl.BlockSpec((1, NC, BN, 128), lambda j, nb: (j, 0, nb, 0)),
            pl.BlockSpec((1, BN, 1), lambda j, nb: (j, nb, 0)),
            pl.BlockSpec((1, 1, D), lambda j, nb: (1, 0, 0)),
        ],
        out_specs=pl.BlockSpec((1, BN, D), lambda j, nb: (j, nb, 0)),
        out_shape=jax.ShapeDtypeStruct((R, N, D), jnp.float32),
    )(s, h, dinv, gcn_b)


def kernel(features, rel_emb, edge_index, gcn_W, gcn_b, bn_gamma, bn_beta,
           lin_W, lin_b, is_training):
    src = edge_index[:, 0, :]
    dst = edge_index[:, 1, :]
    pad = EPAD - E
    srcp = jnp.pad(src, ((0, 0), (0, pad)))
    dstp = jnp.pad(dst, ((0, 0), (0, pad)), constant_values=N)
    dst4 = dstp.reshape(R, NS, NCH, 128)
    offs = (jnp.arange(NC, dtype=jnp.int32) * NH)[:, None, None] + \
           (jnp.arange(R, dtype=jnp.int32) * (NC * NH))[None, :, None]
    srcadj = (srcp[None, :, :] + offs).reshape(NC, R, NS, NCH, 128)
    zblk = jnp.zeros((128, 128), jnp.float32)

    part = _deg_kernel(dst4)
    r1, frs = _rel_call(rel_emb, lin_W, lin_b)

    h0, dinv = _mm1_call(features, rel_emb.reshape(R, 1, D), gcn_W,
                         part.reshape(NC * NS, R, NH, 1))
    s0 = _conv_kernel(h0.reshape(R * NC * NH, 128), srcadj, dst4, zblk)
    stats = _stats_call(s0, h0, dinv)
    h1 = _mm2_call(features, s0, h0, dinv, stats, bn_gamma, bn_beta,
                   r1, gcn_W)
    s1 = _conv_kernel(h1.reshape(R * NC * NH, 128), srcadj, dst4, zblk)
    out3 = _final_call(s1, h1, dinv, gcn_b.reshape(2, 1, D))
    return (out3[0], out3[1], out3[2], frs[0], frs[1], frs[2])


# speedup vs baseline: 1.4581x; 1.4581x over previous
import functools

import jax
import jax.numpy as jnp
from jax import lax
from jax.experimental import pallas as pl
from jax.experimental.pallas import tpu as pltpu
from jax.experimental.pallas import tpu_sc as plsc

N = 10000
D = 256
E = 160000
R = 3
NC = 2
NS = 16
NCH = 80
EPT = NCH * 128
EPAD = EPT * NS
NH = N + 112
RPT = NH // NS
BN = 400
NBLK = N // BN
NPAIR = NCH // 2

_mesh = plsc.VectorSubcoreMesh(
    core_axis_name="c", subcore_axis_name="s", num_cores=NC, num_subcores=NS)

HB = R * NH


@functools.partial(
    pl.kernel,
    out_type=jax.ShapeDtypeStruct((NC * NS, HB), jnp.float32),
    mesh=_mesh,
    scratch_types=[
        pltpu.VMEM((HB,), jnp.float32),
        pltpu.VMEM((NCH // 2, 128), jnp.int32),
    ],
    compiler_params=pltpu.CompilerParams(needs_layout_passes=False),
)
def _deg_kernel(dst_hbm, out_hbm, hist, dstb):
    c = lax.axis_index("c")
    t = lax.axis_index("s")
    wid = c * NS + t
    zeros16 = jnp.zeros((16,), jnp.float32)
    ones16 = jnp.ones((16,), jnp.float32)

    def zbody(i, _):
        hist[pl.ds(i * 16, 16)] = zeros16
        return 0

    lax.fori_loop(0, HB // 16, zbody, 0)
    for j in range(R):
        pltpu.sync_copy(dst_hbm.at[j, t, pl.ds(c * (NCH // 2), NCH // 2)],
                        dstb)

        def body(r, _):
            for k in range(8):
                idx = dstb[r, pl.ds(k * 16, 16)]
                plsc.addupdate_scatter(hist, [idx + j * NH], ones16)
            return 0

        lax.fori_loop(0, NCH // 2, body, 0)
    pltpu.sync_copy(hist, out_hbm.at[wid])


@functools.partial(
    pl.kernel,
    out_type=jax.ShapeDtypeStruct((R, NC, NH, 128), jnp.float32),
    mesh=_mesh,
    scratch_types=[
        pltpu.VMEM_SHARED((NH, 128), jnp.float32),
        pltpu.VMEM((NCH // 2, 128), jnp.int32),
        pltpu.VMEM((NCH // 2, 128), jnp.int32),
        pltpu.VMEM((64, 128), jnp.float32),
        pltpu.VMEM((64, 128), jnp.float32),
        pltpu.VMEM((64, 128), jnp.float32),
        pltpu.VMEM((64, 128), jnp.float32),
        pltpu.SemaphoreType.DMA,
        pltpu.SemaphoreType.DMA,
        pltpu.SemaphoreType.DMA,
        pltpu.SemaphoreType.DMA,
    ],
    compiler_params=pltpu.CompilerParams(needs_layout_passes=False),
)
def _conv_kernel(h_hbm, srcadj_hbm, dst_hbm, z_hbm, s_hbm,
                 acc, srcb, dstb, r0, r1b, r2, r3, s0, s1b, s2, s3):
    c = lax.axis_index("c")
    t = lax.axis_index("s")
    base = t * RPT
    bufs = [(r0, s0), (r1b, s1b), (r2, s2), (r3, s3)]
    for j in range(R):
        for k in range(RPT // 128):
            pltpu.sync_copy(z_hbm, acc.at[pl.ds(base + 128 * k, 128)])
        rem = RPT % 128
        if rem:
            pltpu.sync_copy(z_hbm.at[pl.ds(0, rem)],
                            acc.at[pl.ds(base + RPT - rem, rem)])
        plsc.subcore_barrier()

        def gissue(sc, buf, sem):
            pltpu.async_copy(
                h_hbm.at[srcb.at[sc // 2, pl.ds((sc % 2) * 64, 64)]],
                buf, sem)

        def gwait(buf, sem):
            pltpu.make_async_copy(
                h_hbm.at[srcb.at[0, pl.ds(0, 64)]], buf, sem).wait()

        def scat(sc, buf):
            pltpu.sync_copy(
                buf, acc.at[dstb.at[sc // 2, pl.ds((sc % 2) * 64, 64)]],
                add=True)

        NSC = NCH
        for half in range(2):
            pltpu.sync_copy(
                srcadj_hbm.at[c, j, t, pl.ds(half * (NCH // 2), NCH // 2)],
                srcb)
            pltpu.sync_copy(
                dst_hbm.at[j, t, pl.ds(half * (NCH // 2), NCH // 2)], dstb)
            for b in range(4):
                gissue(b, bufs[b][0], bufs[b][1])

            def quad(i, _):
                q0 = 4 * i
                for b in range(4):
                    gwait(bufs[b][0], bufs[b][1])
                    scat(q0 + b, bufs[b][0])

                    @pl.when(i < NSC // 4 - 1)
                    def _():
                        gissue(q0 + 4 + b, bufs[b][0], bufs[b][1])
                return 0

            lax.fori_loop(0, NSC // 4, quad, 0)
        plsc.subcore_barrier()
        pltpu.sync_copy(acc.at[pl.ds(base, RPT)],
                        s_hbm.at[j, c, pl.ds(base, RPT)])
        plsc.subcore_barrier()


def _params_body(part_ref, rel_ref, lw_ref, lb_ref, dinv_ref, r1_ref, fr_ref):
    deg = jnp.sum(part_ref[...], axis=0) + 1.0
    dinv_ref[...] = lax.rsqrt(deg).reshape(R, NH, 1)
    r1 = jnp.dot(rel_ref[...], lw_ref[0].T,
                 preferred_element_type=jnp.float32) + lb_ref[0]
    r1_ref[...] = r1.reshape(R, 1, D)
    fr_ref[...] = jnp.dot(r1, lw_ref[1].T,
                          preferred_element_type=jnp.float32) + lb_ref[1]


def _params_call(part, rel_emb, lin_W, lin_b):
    return pl.pallas_call(
        _params_body,
        out_shape=(
            jax.ShapeDtypeStruct((R, NH, 1), jnp.float32),
            jax.ShapeDtypeStruct((R, 1, D), jnp.float32),
            jax.ShapeDtypeStruct((R, D), jnp.float32),
        ),
    )(part, rel_emb, lin_W, lin_b)


def _mm1_body(f_ref, rel_ref, w_ref, dinv_ref, out_ref):
    x = f_ref[...] * rel_ref[0]
    dv = dinv_ref[0]
    w = w_ref[0]
    out_ref[0, 0] = jnp.dot(x, w[:, :128],
                            preferred_element_type=jnp.float32) * dv
    out_ref[0, 1] = jnp.dot(x, w[:, 128:],
                            preferred_element_type=jnp.float32) * dv


def _mm1_call(features, rel_emb, gcn_W, dinv):
    return pl.pallas_call(
        _mm1_body,
        grid=(R, NBLK),
        in_specs=[
            pl.BlockSpec((BN, D), lambda j, nb: (nb, 0)),
            pl.BlockSpec((1, 1, D), lambda j, nb: (j, 0, 0)),
            pl.BlockSpec((1, D, D), lambda j, nb: (0, 0, 0)),
            pl.BlockSpec((1, BN, 1), lambda j, nb: (j, nb, 0)),
        ],
        out_specs=pl.BlockSpec((1, NC, BN, 128), lambda j, nb: (j, 0, nb, 0)),
        out_shape=jax.ShapeDtypeStruct((R, NC, NH, 128), jnp.float32),
    )(features, rel_emb, gcn_W, dinv)


def _stats_body(s_ref, h_ref, dinv_ref, out_ref):
    nb = pl.program_id(1)
    dv = dinv_ref[0]
    acc = []
    for cc in range(NC):
        tt = dv * (s_ref[0, cc] + h_ref[0, cc])
        acc.append(jnp.stack([jnp.sum(tt, axis=0), jnp.sum(tt * tt, axis=0)]))
    v = jnp.stack(acc)

    @pl.when(nb == 0)
    def _():
        out_ref[0] = v

    @pl.when(nb > 0)
    def _():
        out_ref[0] += v


def _stats_call(s, h, dinv):
    return pl.pallas_call(
        _stats_body,
        grid=(R, NBLK),
        in_specs=[
            pl.BlockSpec((1, NC, BN, 128), lambda j, nb: (j, 0, nb, 0)),
            pl.BlockSpec((1, NC, BN, 128), lambda j, nb: (j, 0, nb, 0)),
            pl.BlockSpec((1, BN, 1), lambda j, nb: (j, nb, 0)),
        ],
        out_specs=pl.BlockSpec((1, NC, 2, 128), lambda j, nb: (j, 0, 0, 0)),
        out_shape=jax.ShapeDtypeStruct((R, NC, 2, 128), jnp.float32),
    )(s, h, dinv)


def _mm2_body(f_ref, s_ref, h_ref, dinv_ref, st_ref, g_ref, be_ref,
              r1_ref, w_ref, out_ref):
    dv = dinv_ref[0]
    t = jnp.concatenate(
        [dv * (s_ref[0, cc] + h_ref[0, cc]) for cc in range(NC)], axis=1)
    ssum = jnp.concatenate([st_ref[0, cc, 0] for cc in range(NC)])
    ssq = jnp.concatenate([st_ref[0, cc, 1] for cc in range(NC)])
    mu = ssum / N
    var = ssq / N - mu * mu
    bh = (t - mu[None, :]) * lax.rsqrt(var + 1e-5)[None, :]
    bh = bh * g_ref[0][None, :] + be_ref[0][None, :]
    x1 = (f_ref[...] + jnp.where(bh >= 0.0, bh, 0.01 * bh)) * r1_ref[0]
    w = w_ref[0]
    out_ref[0, 0] = jnp.dot(x1, w[:, :128],
                            preferred_element_type=jnp.float32) * dv
    out_ref[0, 1] = jnp.dot(x1, w[:, 128:],
                            preferred_element_type=jnp.float32) * dv


def _mm2_call(features, s, h, dinv, stats, bn_gamma, bn_beta, r1, gcn_W):
    return pl.pallas_call(
        _mm2_body,
        grid=(R, NBLK),
        in_specs=[
            pl.BlockSpec((BN, D), lambda j, nb: (nb, 0)),
            pl.BlockSpec((1, NC, BN, 128), lambda j, nb: (j, 0, nb, 0)),
            pl.BlockSpec((1, NC, BN, 128), lambda j, nb: (j, 0, nb, 0)),
            pl.BlockSpec((1, BN, 1), lambda j, nb: (j, nb, 0)),
            pl.BlockSpec((1, NC, 2, 128), lambda j, nb: (j, 0, 0, 0)),
            pl.BlockSpec((1, D), lambda j, nb: (0, 0)),
            pl.BlockSpec((1, D), lambda j, nb: (0, 0)),
            pl.BlockSpec((1, 1, D), lambda j, nb: (j, 0, 0)),
            pl.BlockSpec((1, D, D), lambda j, nb: (1, 0, 0)),
        ],
        out_specs=pl.BlockSpec((1, NC, BN, 128), lambda j, nb: (j, 0, nb, 0)),
        out_shape=jax.ShapeDtypeStruct((R, NC, NH, 128), jnp.float32),
    )(features, s, h, dinv, stats, bn_gamma, bn_beta, r1, gcn_W)


def _final_body(s_ref, h_ref, dinv_ref, b_ref, out_ref):
    dv = dinv_ref[0]
    t = jnp.concatenate(
        [dv * (s_ref[0, cc] + h_ref[0, cc]) for cc in range(NC)], axis=1)
    out_ref[0] = t + b_ref[0]


def _final_call(s, h, dinv, gcn_b):
    return pl.pallas_call(
        _final_body,
        grid=(R, NBLK),
        in_specs=[
            pl.BlockSpec((1, NC, BN, 128), lambda j, nb: (j, 0, nb, 0)),
            pl.BlockSpec((1, NC, BN, 128), lambda j, nb: (j, 0, nb, 0)),
            pl.BlockSpec((1, BN, 1), lambda j, nb: (j, nb, 0)),
            pl.BlockSpec((1, 1, D), lambda j, nb: (1, 0, 0)),
        ],
        out_specs=pl.BlockSpec((1, BN, D), lambda j, nb: (j, nb, 0)),
        out_shape=jax.ShapeDtypeStruct((R, N, D), jnp.float32),
    )(s, h, dinv, gcn_b)


def kernel(features, rel_emb, edge_index, gcn_W, gcn_b, bn_gamma, bn_beta,
           lin_W, lin_b, is_training):
    src = edge_index[:, 0, :]
    dst = edge_index[:, 1, :]
    pad = EPAD - E
    srcp = jnp.pad(src, ((0, 0), (0, pad)))
    dstp = jnp.pad(dst, ((0, 0), (0, pad)), constant_values=N)
    dst4 = dstp.reshape(R, NS, NCH, 128)
    offs = (jnp.arange(NC, dtype=jnp.int32) * NH)[:, None, None] + \
           (jnp.arange(R, dtype=jnp.int32) * (NC * NH))[None, :, None]
    srcadj = (srcp[None, :, :] + offs).reshape(NC, R, NS, NCH, 128)
    zblk = jnp.zeros((128, 128), jnp.float32)

    part = _deg_kernel(dst4)
    dinv, r1, frs = _params_call(part.reshape(NC * NS, R, NH),
                                 rel_emb, lin_W, lin_b)

    h0 = _mm1_call(features, rel_emb.reshape(R, 1, D), gcn_W, dinv)
    s0 = _conv_kernel(h0.reshape(R * NC * NH, 128), srcadj, dst4, zblk)
    stats = _stats_call(s0, h0, dinv)
    h1 = _mm2_call(features, s0, h0, dinv, stats, bn_gamma, bn_beta,
                   r1, gcn_W)
    s1 = _conv_kernel(h1.reshape(R * NC * NH, 128), srcadj, dst4, zblk)
    out3 = _final_call(s1, h1, dinv, gcn_b.reshape(2, 1, D))
    return (out3[0], out3[1], out3[2], frs[0], frs[1], frs[2])
